# baseline (speedup 1.0000x reference)
import functools

import jax
import jax.numpy as jnp
from jax import lax
from jax.experimental import pallas as pl
from jax.experimental.pallas import tpu as pltpu
from jax.experimental.pallas import tpu_sc as plsc

NUM_NODES = 1000000
D = 64
B = 16384
NC = 2
NS = 16
NW = NC * NS
BPW = B // NW
HALF = BPW // 2
TCOL = 16384
TBITS = TCOL.bit_length() - 1


def _sc_body(table2_hbm, i_hbm, j_hbm, n_hbm, dummy_hbm, pos_hbm, neg_hbm,
             ii_v, ij_v, in_v, ui_v, uj_v, un_v, pp_v, pn_v, pos_v, neg_v,
             sem):
    wid = lax.axis_index("s") * NC + lax.axis_index("c")
    base = wid * BPW

    for src, dst in ((i_hbm, ii_v), (j_hbm, ij_v), (n_hbm, in_v)):
        pltpu.sync_copy(src.at[pl.ds(base, BPW)], dst)

    for h in range(2):
        for idx, dst in ((ii_v, ui_v), (ij_v, uj_v), (in_v, un_v)):
            def issue(g, _, h=h, idx=idx, dst=dst):
                rows = idx[pl.ds(h * HALF + g * 16, 16)]
                pairs = ((rows >> (TBITS + 1)) << TBITS) + (rows & (TCOL - 1))
                for k in range(16):
                    pltpu.make_async_copy(
                        table2_hbm.at[pl.ds(pairs[k], 1), :],
                        dst.at[pl.ds(g * 16 + k, 1), :],
                        sem).start()
                return 0

            lax.fori_loop(0, HALF // 16, issue, 0)
        for dst in (ui_v, uj_v, un_v):
            pltpu.make_async_copy(dummy_hbm, dst, sem).wait()

        def row_body(g, _, h=h):
            pi = ((ii_v[pl.ds(h * HALF + g * 16, 16)] >> TBITS) & 1) * 32
            pj = ((ij_v[pl.ds(h * HALF + g * 16, 16)] >> TBITS) & 1) * 32
            pn_ = ((in_v[pl.ds(h * HALF + g * 16, 16)] >> TBITS) & 1) * 32
            for k in range(16):
                lr = g * 16 + k
                accp = jnp.zeros((16,), jnp.float32)
                accn = jnp.zeros((16,), jnp.float32)
                for kk in range(2):
                    vi = ui_v[lr, pl.ds(pi[k] + kk * 16, 16)]
                    vj = uj_v[lr, pl.ds(pj[k] + kk * 16, 16)]
                    vn = un_v[lr, pl.ds(pn_[k] + kk * 16, 16)]
                    ia, ib = plsc.unpack(plsc.bitcast(vi, jnp.bfloat16),
                                         format=plsc.PackFormat.INTERLEAVED)
                    ja, jb = plsc.unpack(plsc.bitcast(vj, jnp.bfloat16),
                                         format=plsc.PackFormat.INTERLEAVED)
                    na, nb = plsc.unpack(plsc.bitcast(vn, jnp.bfloat16),
                                         format=plsc.PackFormat.INTERLEAVED)
                    accp = accp + ia * ja + ib * jb
                    accn = accn + ia * na + ib * nb
                pp_v[pl.ds((h * HALF + lr) * 16, 16)] = accp
                pn_v[pl.ds((h * HALF + lr) * 16, 16)] = accn
            return 0

        lax.fori_loop(0, HALF // 16, row_body, 0)

    lanes = lax.iota(jnp.int32, 16)

    def group_body(g, _):
        base_ids = (g * 16 + lanes) * 16

        def l_body(l, acc):
            accp, accn = acc
            ids = base_ids + l
            accp = accp + plsc.load_gather(pp_v, [ids])
            accn = accn + plsc.load_gather(pn_v, [ids])
            return accp, accn

        zeros = jnp.zeros((16,), jnp.float32)
        accp, accn = lax.fori_loop(0, 16, l_body, (zeros, zeros))
        pos_v[pl.ds(g * 16, 16)] = accp
        neg_v[pl.ds(g * 16, 16)] = accn
        return 0

    lax.fori_loop(0, BPW // 16, group_body, 0)

    pltpu.sync_copy(pos_v, pos_hbm.at[pl.ds(base, BPW)])
    pltpu.sync_copy(neg_v, neg_hbm.at[pl.ds(base, BPW)])


@jax.jit
def _sc_dots(table2, i, j, neg_j):
    mesh = plsc.VectorSubcoreMesh(core_axis_name="c", subcore_axis_name="s")
    return pl.kernel(
        _sc_body,
        out_type=(jax.ShapeDtypeStruct((B,), jnp.float32),
                  jax.ShapeDtypeStruct((B,), jnp.float32)),
        mesh=mesh,
        compiler_params=pltpu.CompilerParams(needs_layout_passes=False),
        scratch_types=[
            pltpu.VMEM((BPW,), jnp.int32),
            pltpu.VMEM((BPW,), jnp.int32),
            pltpu.VMEM((BPW,), jnp.int32),
            pltpu.VMEM((HALF, D), jnp.float32),
            pltpu.VMEM((HALF, D), jnp.float32),
            pltpu.VMEM((HALF, D), jnp.float32),
            pltpu.VMEM((BPW * 16,), jnp.float32),
            pltpu.VMEM((BPW * 16,), jnp.float32),
            pltpu.VMEM((BPW,), jnp.float32),
            pltpu.VMEM((BPW,), jnp.float32),
            pltpu.SemaphoreType.DMA,
        ],
    )(table2, i, j, neg_j, jnp.zeros((HALF, D), jnp.float32))


def _loss_body(pos_ref, neg_ref, out_ref):
    p = pos_ref[...]
    n = neg_ref[...]

    def softplus(z):
        return jnp.maximum(z, 0.0) + jnp.log1p(jnp.exp(-jnp.abs(z)))

    loss = softplus(-p) + softplus(n)
    out_ref[0, 0] = jnp.sum(loss) / B


@jax.jit
def _tc_loss(pos, neg):
    out = pl.pallas_call(
        _loss_body,
        out_shape=jax.ShapeDtypeStruct((1, 1), jnp.float32),
        out_specs=pl.BlockSpec(memory_space=pltpu.SMEM),
    )(pos.reshape(128, 128), neg.reshape(128, 128))
    return out[0, 0]


TGRID = -(-NUM_NODES // (2 * TCOL))
TROWS = TGRID * TCOL


def _transpose_body(a_ref, out_ref):
    stacked = jnp.concatenate(
        [a_ref[:, 0:TCOL], a_ref[:, TCOL:2 * TCOL]], axis=0)
    xt = stacked.T
    u = lax.bitcast_convert_type(xt, jnp.uint32)

    def rnd(v):
        return (v + 0x7FFF + ((v >> 16) & 1)) >> 16

    lo = rnd(jnp.concatenate([u[:, 0:32], u[:, 64:96]], axis=1))
    hi = rnd(jnp.concatenate([u[:, 32:64], u[:, 96:128]], axis=1))
    out_ref[...] = lax.bitcast_convert_type(lo | (hi << 16), jnp.float32)


@jax.jit
def _tc_transpose(tableT):
    return pl.pallas_call(
        _transpose_body,
        grid=(TGRID,),
        in_specs=[pl.BlockSpec((D, 2 * TCOL), lambda c: (0, c))],
        out_specs=pl.BlockSpec((TCOL, D), lambda c: (c, 0)),
        out_shape=jax.ShapeDtypeStruct((TROWS, D), jnp.float32),
    )(tableT)


def kernel(table, i, j, neg_j):
    i = i.astype(jnp.int32)
    j = j.astype(jnp.int32)
    neg_j = neg_j.astype(jnp.int32)
    table2 = _tc_transpose(jnp.swapaxes(table, 0, 1))
    pos, neg = _sc_dots(table2, i, j, neg_j)
    return _tc_loss(pos, neg)


# speedup vs baseline: 1.2765x; 1.2765x over previous
import functools

import jax
import jax.numpy as jnp
from jax import lax
from jax.experimental import pallas as pl
from jax.experimental.pallas import tpu as pltpu
from jax.experimental.pallas import tpu_sc as plsc

NUM_NODES = 1000000
D = 64
B = 16384
NC = 2
NS = 16
NW = NC * NS
BPW = B // NW
HALF = BPW // 2
TCOL = 16384
TBITS = TCOL.bit_length() - 1


def _sc_body(table2_hbm, i_hbm, j_hbm, n_hbm, dummy_hbm, pos_hbm, neg_hbm,
             ii_v, ij_v, in_v, ui_v, uj_v, un_v, pp_v, pn_v, pos_v, neg_v,
             sem):
    wid = lax.axis_index("s") * NC + lax.axis_index("c")
    base = wid * BPW

    for src, dst in ((i_hbm, ii_v), (j_hbm, ij_v), (n_hbm, in_v)):
        pltpu.sync_copy(src.at[pl.ds(base, BPW)], dst)

    for h in range(2):
        for idx, dst in ((ii_v, ui_v), (ij_v, uj_v), (in_v, un_v)):
            def issue(g, _, h=h, idx=idx, dst=dst):
                rows = idx[pl.ds(h * HALF + g * 16, 16)]
                pairs = ((rows >> (TBITS + 1)) << TBITS) + (rows & (TCOL - 1))
                for k in range(16):
                    pltpu.make_async_copy(
                        table2_hbm.at[pl.ds(pairs[k], 1), :],
                        dst.at[pl.ds(g * 16 + k, 1), :],
                        sem).start()
                return 0

            lax.fori_loop(0, HALF // 16, issue, 0)
        for dst in (ui_v, uj_v, un_v):
            pltpu.make_async_copy(dummy_hbm, dst, sem).wait()

        def row_body(g, _, h=h):
            pi = (ii_v[pl.ds(h * HALF + g * 16, 16)] >> TBITS) & 1
            pj = (ij_v[pl.ds(h * HALF + g * 16, 16)] >> TBITS) & 1
            pn_ = (in_v[pl.ds(h * HALF + g * 16, 16)] >> TBITS) & 1
            for k in range(16):
                oi = pi[k] * D
                oj = pj[k] * D
                on = pn_[k] * D
                lr = g * 16 + k
                accp = jnp.zeros((16,), jnp.float32)
                accn = jnp.zeros((16,), jnp.float32)
                for kk in range(D // 16):
                    vi = ui_v[lr, pl.ds(oi + kk * 16, 16)]
                    vj = uj_v[lr, pl.ds(oj + kk * 16, 16)]
                    vn = un_v[lr, pl.ds(on + kk * 16, 16)]
                    accp = accp + vi * vj
                    accn = accn + vi * vn
                pp_v[pl.ds((h * HALF + lr) * 16, 16)] = accp
                pn_v[pl.ds((h * HALF + lr) * 16, 16)] = accn
            return 0

        lax.fori_loop(0, HALF // 16, row_body, 0)

    lanes = lax.iota(jnp.int32, 16)

    def group_body(g, _):
        base_ids = (g * 16 + lanes) * 16

        def l_body(l, acc):
            accp, accn = acc
            ids = base_ids + l
            accp = accp + plsc.load_gather(pp_v, [ids])
            accn = accn + plsc.load_gather(pn_v, [ids])
            return accp, accn

        zeros = jnp.zeros((16,), jnp.float32)
        accp, accn = lax.fori_loop(0, 16, l_body, (zeros, zeros))
        pos_v[pl.ds(g * 16, 16)] = accp
        neg_v[pl.ds(g * 16, 16)] = accn
        return 0

    lax.fori_loop(0, BPW // 16, group_body, 0)

    pltpu.sync_copy(pos_v, pos_hbm.at[pl.ds(base, BPW)])
    pltpu.sync_copy(neg_v, neg_hbm.at[pl.ds(base, BPW)])


@jax.jit
def _sc_dots(table2, i, j, neg_j):
    mesh = plsc.VectorSubcoreMesh(core_axis_name="c", subcore_axis_name="s")
    return pl.kernel(
        _sc_body,
        out_type=(jax.ShapeDtypeStruct((B,), jnp.float32),
                  jax.ShapeDtypeStruct((B,), jnp.float32)),
        mesh=mesh,
        compiler_params=pltpu.CompilerParams(needs_layout_passes=False),
        scratch_types=[
            pltpu.VMEM((BPW,), jnp.int32),
            pltpu.VMEM((BPW,), jnp.int32),
            pltpu.VMEM((BPW,), jnp.int32),
            pltpu.VMEM((HALF, 2 * D), jnp.float32),
            pltpu.VMEM((HALF, 2 * D), jnp.float32),
            pltpu.VMEM((HALF, 2 * D), jnp.float32),
            pltpu.VMEM((BPW * 16,), jnp.float32),
            pltpu.VMEM((BPW * 16,), jnp.float32),
            pltpu.VMEM((BPW,), jnp.float32),
            pltpu.VMEM((BPW,), jnp.float32),
            pltpu.SemaphoreType.DMA,
        ],
    )(table2, i, j, neg_j, jnp.zeros((HALF, 2 * D), jnp.float32))


def _loss_body(pos_ref, neg_ref, out_ref):
    p = pos_ref[...]
    n = neg_ref[...]

    def softplus(z):
        return jnp.maximum(z, 0.0) + jnp.log1p(jnp.exp(-jnp.abs(z)))

    loss = softplus(-p) + softplus(n)
    out_ref[0, 0] = jnp.sum(loss) / B


@jax.jit
def _tc_loss(pos, neg):
    out = pl.pallas_call(
        _loss_body,
        out_shape=jax.ShapeDtypeStruct((1, 1), jnp.float32),
        out_specs=pl.BlockSpec(memory_space=pltpu.SMEM),
    )(pos.reshape(128, 128), neg.reshape(128, 128))
    return out[0, 0]


TGRID = -(-NUM_NODES // (2 * TCOL))
TROWS = TGRID * TCOL


def _transpose_body(a_ref, out_ref):
    stacked = jnp.concatenate(
        [a_ref[:, 0:TCOL], a_ref[:, TCOL:2 * TCOL]], axis=0)
    out_ref[...] = stacked.T


@jax.jit
def _tc_transpose(tableT):
    return pl.pallas_call(
        _transpose_body,
        grid=(TGRID,),
        in_specs=[pl.BlockSpec((D, 2 * TCOL), lambda c: (0, c))],
        out_specs=pl.BlockSpec((TCOL, 2 * D), lambda c: (c, 0)),
        out_shape=jax.ShapeDtypeStruct((TROWS, 2 * D), jnp.float32),
    )(tableT)


def kernel(table, i, j, neg_j):
    i = i.astype(jnp.int32)
    j = j.astype(jnp.int32)
    neg_j = neg_j.astype(jnp.int32)
    table2 = _tc_transpose(jnp.swapaxes(table, 0, 1))
    pos, neg = _sc_dots(table2, i, j, neg_j)
    return _tc_loss(pos, neg)
